# baseline (speedup 1.0000x reference)
import functools
import jax
import jax.numpy as jnp
from jax import lax
from jax.experimental import pallas as pl
from jax.experimental.pallas import tpu as pltpu
from jax.experimental.pallas import tpu_sc as plsc

TOKEN_BLOCK = 1024
ROUTER_TOP_K = 2
ROUTER_CAPACITY_FACTOR = 1.0


def _router_block(x_ref, w_ref, disp_ref, gates_ref, eidx_ref, carry_ref,
                  *, capacity, num_experts):
    T = x_ref.shape[0]
    E = num_experts

    @pl.when(pl.program_id(0) == 0)
    def _init():
        carry_ref[...] = jnp.zeros_like(carry_ref)

    logits = jnp.dot(x_ref[...], w_ref[...],
                     preferred_element_type=jnp.float32)

    iota_e = lax.broadcasted_iota(jnp.int32, (T, E), 1)
    m0 = jnp.max(logits, axis=1, keepdims=True)
    e0 = jnp.min(jnp.where(logits == m0, iota_e, E), axis=1, keepdims=True)
    mask0 = iota_e == e0
    l2 = jnp.where(mask0, -jnp.inf, logits)
    m1 = jnp.max(l2, axis=1, keepdims=True)
    e1 = jnp.min(jnp.where(l2 == m1, iota_e, E), axis=1, keepdims=True)
    mask1 = iota_e == e1

    t = jnp.exp(m1 - m0)
    denom = 1.0 + t
    gates_ref[...] = jnp.concatenate([1.0 / denom, t / denom], axis=1)
    eidx_ref[...] = jnp.concatenate([e0, e1], axis=1)

    ir = lax.broadcasted_iota(jnp.int32, (T, T), 0)
    ic = lax.broadcasted_iota(jnp.int32, (T, T), 1)
    tri = (ir >= ic).astype(jnp.float32)
    m0f = mask0.astype(jnp.float32)
    m1f = mask1.astype(jnp.float32)
    c0 = jnp.dot(tri, m0f, preferred_element_type=jnp.float32)
    c1 = jnp.dot(tri, m1f, preferred_element_type=jnp.float32)
    p0 = c0 + carry_ref[0:1, :]
    p1 = c1 + carry_ref[1:2, :]
    carry_ref[0:1, :] = carry_ref[0:1, :] + c0[T - 1:T, :]
    carry_ref[1:2, :] = carry_ref[1:2, :] + c1[T - 1:T, :]

    pos0 = jnp.sum(m0f * p0, axis=1, keepdims=True) - 1.0
    pos1 = jnp.sum(m1f * p1, axis=1, keepdims=True) - 1.0
    col0 = e0 * capacity + pos0.astype(jnp.int32)
    col1 = e1 * capacity + pos1.astype(jnp.int32)
    col0 = jnp.where(pos0 < capacity, col0, -1)
    col1 = jnp.where(pos1 < capacity, col1, -1)
    e_iota = lax.broadcasted_iota(jnp.int32, (T, E, capacity), 1)
    c_iota = lax.broadcasted_iota(jnp.int32, (T, E, capacity), 2)
    col3 = e_iota * capacity + c_iota
    d = (col3 == col0[:, :, None]) | (col3 == col1[:, :, None])
    disp_ref[...] = d.astype(jnp.float32)


def _make_sc_copy(nt, d):
    info = plsc.get_sparse_core_info()
    nw = info.num_cores * info.num_subcores
    rows_per = nt // nw
    chunk = 64
    mesh = plsc.VectorSubcoreMesh(core_axis_name="c", subcore_axis_name="s")

    def _sc_copy_body(x_hbm, out_hbm, buf):
        wid = lax.axis_index("s") * info.num_cores + lax.axis_index("c")
        base = wid * rows_per
        for j in range(rows_per // chunk):
            pltpu.sync_copy(x_hbm.at[pl.ds(base + j * chunk, chunk)], buf)
            pltpu.sync_copy(buf, out_hbm.at[pl.ds(base + j * chunk, chunk)])

    return pl.kernel(
        _sc_copy_body,
        out_type=jax.ShapeDtypeStruct((nt, d), jnp.float32),
        mesh=mesh,
        scratch_types=[pltpu.VMEM((chunk, d), jnp.float32)],
    )


def kernel(x, W):
    b, s, d = x.shape
    e = W.shape[1]
    nt = b * s
    capacity = int(nt / e * ROUTER_CAPACITY_FACTOR)
    T = TOKEN_BLOCK
    xf = x.reshape(nt, d)
    disp, gates, eidx = pl.pallas_call(
        functools.partial(_router_block, capacity=capacity, num_experts=e),
        grid=(nt // T,),
        in_specs=[
            pl.BlockSpec((T, d), lambda i: (i, 0)),
            pl.BlockSpec((d, e), lambda i: (0, 0)),
        ],
        out_specs=[
            pl.BlockSpec((T, e, capacity), lambda i: (i, 0, 0)),
            pl.BlockSpec((T, ROUTER_TOP_K), lambda i: (i, 0)),
            pl.BlockSpec((T, ROUTER_TOP_K), lambda i: (i, 0)),
        ],
        out_shape=[
            jax.ShapeDtypeStruct((nt, e, capacity), jnp.float32),
            jax.ShapeDtypeStruct((nt, ROUTER_TOP_K), jnp.float32),
            jax.ShapeDtypeStruct((nt, ROUTER_TOP_K), jnp.int32),
        ],
        scratch_shapes=[pltpu.VMEM((ROUTER_TOP_K, e), jnp.float32)],
        compiler_params=pltpu.CompilerParams(
            dimension_semantics=("arbitrary",),
        ),
    )(xf, W)
    sc_out = _make_sc_copy(nt, d)(xf)
    gates = gates + sc_out[0, 0] * 1e-30
    return (
        disp,
        gates.reshape(b, s, ROUTER_TOP_K),
        eidx.reshape(b, s, ROUTER_TOP_K),
    )


# speedup vs baseline: 1.5655x; 1.5655x over previous
import functools
import jax
import jax.numpy as jnp
from jax import lax
from jax.experimental import pallas as pl
from jax.experimental.pallas import tpu as pltpu

TOKEN_BLOCK = 1024
ROUTER_TOP_K = 2
ROUTER_CAPACITY_FACTOR = 1.0


def _router_block(x_ref, w_ref, disp_ref, gates_ref, eidx_ref, carry_ref,
                  *, capacity, num_experts):
    T = x_ref.shape[0]
    E = num_experts

    @pl.when(pl.program_id(0) == 0)
    def _init():
        carry_ref[...] = jnp.zeros_like(carry_ref)

    logits = jnp.dot(x_ref[...], w_ref[...],
                     preferred_element_type=jnp.float32)

    iota_e = lax.broadcasted_iota(jnp.int32, (T, E), 1)
    m0 = jnp.max(logits, axis=1, keepdims=True)
    e0 = jnp.min(jnp.where(logits == m0, iota_e, E), axis=1, keepdims=True)
    mask0 = iota_e == e0
    l2 = jnp.where(mask0, -jnp.inf, logits)
    m1 = jnp.max(l2, axis=1, keepdims=True)
    e1 = jnp.min(jnp.where(l2 == m1, iota_e, E), axis=1, keepdims=True)
    mask1 = iota_e == e1

    t = jnp.exp(m1 - m0)
    denom = 1.0 + t
    gates_ref[...] = jnp.concatenate([1.0 / denom, t / denom], axis=1)
    eidx_ref[...] = jnp.concatenate([e0, e1], axis=1)

    ir = lax.broadcasted_iota(jnp.int32, (T, T), 0)
    ic = lax.broadcasted_iota(jnp.int32, (T, T), 1)
    tri = (ir >= ic).astype(jnp.float32)
    m0f = mask0.astype(jnp.float32)
    m1f = mask1.astype(jnp.float32)
    c0 = jnp.dot(tri, m0f, preferred_element_type=jnp.float32)
    c1 = jnp.dot(tri, m1f, preferred_element_type=jnp.float32)
    p0 = c0 + carry_ref[0:1, :]
    p1 = c1 + carry_ref[1:2, :]
    carry_ref[0:1, :] = carry_ref[0:1, :] + c0[T - 1:T, :]
    carry_ref[1:2, :] = carry_ref[1:2, :] + c1[T - 1:T, :]

    pos0 = jnp.sum(m0f * p0, axis=1, keepdims=True) - 1.0
    pos1 = jnp.sum(m1f * p1, axis=1, keepdims=True) - 1.0
    col0 = e0 * capacity + pos0.astype(jnp.int32)
    col1 = e1 * capacity + pos1.astype(jnp.int32)
    col0 = jnp.where(pos0 < capacity, col0, -1)
    col1 = jnp.where(pos1 < capacity, col1, -1)
    e_iota = lax.broadcasted_iota(jnp.int32, (T, E, capacity), 1)
    c_iota = lax.broadcasted_iota(jnp.int32, (T, E, capacity), 2)
    col3 = e_iota * capacity + c_iota
    d = (col3 == col0[:, :, None]) | (col3 == col1[:, :, None])
    disp_ref[...] = d.astype(jnp.float32)


def kernel(x, W):
    b, s, d = x.shape
    e = W.shape[1]
    nt = b * s
    capacity = int(nt / e * ROUTER_CAPACITY_FACTOR)
    T = TOKEN_BLOCK
    xf = x.reshape(nt, d)
    disp, gates, eidx = pl.pallas_call(
        functools.partial(_router_block, capacity=capacity, num_experts=e),
        grid=(nt // T,),
        in_specs=[
            pl.BlockSpec((T, d), lambda i: (i, 0)),
            pl.BlockSpec((d, e), lambda i: (0, 0)),
        ],
        out_specs=[
            pl.BlockSpec((T, e, capacity), lambda i: (i, 0, 0)),
            pl.BlockSpec((T, ROUTER_TOP_K), lambda i: (i, 0)),
            pl.BlockSpec((T, ROUTER_TOP_K), lambda i: (i, 0)),
        ],
        out_shape=[
            jax.ShapeDtypeStruct((nt, e, capacity), jnp.float32),
            jax.ShapeDtypeStruct((nt, ROUTER_TOP_K), jnp.float32),
            jax.ShapeDtypeStruct((nt, ROUTER_TOP_K), jnp.int32),
        ],
        scratch_shapes=[pltpu.VMEM((ROUTER_TOP_K, e), jnp.float32)],
        compiler_params=pltpu.CompilerParams(
            dimension_semantics=("arbitrary",),
        ),
    )(xf, W)
    return (
        disp,
        gates.reshape(b, s, ROUTER_TOP_K),
        eidx.reshape(b, s, ROUTER_TOP_K),
    )
